# baseline (speedup 1.0000x reference)
import jax
import jax.numpy as jnp
from jax import lax
from jax.experimental import pallas as pl
from jax.experimental.pallas import tpu as pltpu
from jax.experimental.pallas import tpu_sc as plsc

N = 10000
E = 320000
D = 128

NC = 2
NS = 16
NW = NC * NS
K = 64
CPT = 158
EPT = K * CPT
EPAD = EPT * NW
RPS = N // NS
NEG_SLOPE = 0.01

_SC_PARAMS = pltpu.CompilerParams(needs_layout_passes=False)


def _tc1_body(h_ref, wt_ref, aw_ref, z_ref, sl_ref, sr_ref):
    z = jnp.dot(h_ref[...], wt_ref[...], preferred_element_type=jnp.float32)
    z_ref[...] = z
    al = aw_ref[0, :D]
    ar = aw_ref[0, D:]
    sl_ref[...] = jnp.sum(z * al[None, :], axis=1)
    sr_ref[...] = jnp.sum(z * ar[None, :], axis=1)


def _tc1(h, wt, a_w):
    return pl.pallas_call(
        _tc1_body,
        out_shape=[
            jax.ShapeDtypeStruct((N, D), jnp.float32),
            jax.ShapeDtypeStruct((N,), jnp.float32),
            jax.ShapeDtypeStruct((N,), jnp.float32),
        ],
    )(h, wt, a_w)


def _sca_body(sl_hbm, sr_hbm, ef_hbm, zden_hbm,
              ex_hbm, den_hbm,
              sl_v, sr_v, src_v, dst_v, ex_v, den_v, mbuf_v):
    cid = lax.axis_index("c")
    sid = lax.axis_index("s")
    wid = sid * NC + cid

    pltpu.sync_copy(sl_hbm, sl_v)
    pltpu.sync_copy(sr_hbm, sr_v)
    pltpu.sync_copy(ef_hbm.at[0, wid], src_v)
    pltpu.sync_copy(ef_hbm.at[1, wid], dst_v)
    pltpu.sync_copy(zden_hbm, den_v)

    iota16 = lax.iota(jnp.int32, 16)

    def _mx(ref):
        def body(i, m):
            return jnp.maximum(m, ref[pl.ds(i * 16, 16)])
        m16 = lax.fori_loop(0, N // 16, body,
                            jnp.full((16,), -jnp.inf, jnp.float32))
        for sh in (8, 4, 2, 1):
            mbuf_v[pl.ds(0, 16)] = m16
            m16 = jnp.maximum(m16, plsc.load_gather(
                mbuf_v, [jnp.bitwise_xor(iota16, sh)]))
        return m16

    msum = _mx(sl_v) + _mx(sr_v)
    M = jnp.where(msum > 0, msum, NEG_SLOPE * msum)

    ebase = wid * EPT

    def pa(i, carry):
        s16 = src_v[pl.ds(i * 16, 16)]
        d16 = dst_v[pl.ds(i * 16, 16)]
        a = plsc.load_gather(sl_v, [s16])
        b = plsc.load_gather(sr_v, [d16])
        e = a + b
        e = jnp.where(e > 0, e, NEG_SLOPE * e)
        ex = jnp.exp(e - M)
        gid = ebase + i * 16 + iota16
        ex = jnp.where(gid < E, ex, 0.0)
        ex_v[pl.ds(i * 16, 16)] = ex
        plsc.addupdate_scatter(den_v, [d16], ex)
        return carry

    lax.fori_loop(0, EPT // 16, pa, 0)
    pltpu.sync_copy(ex_v, ex_hbm.at[wid])
    pltpu.sync_copy(den_v, den_hbm.at[wid])


def _sca(sl, sr, eflat, zden):
    mesh = plsc.VectorSubcoreMesh(core_axis_name="c", subcore_axis_name="s")
    f = pl.kernel(
        _sca_body,
        out_type=[
            jax.ShapeDtypeStruct((NW, EPT), jnp.float32),
            jax.ShapeDtypeStruct((NW, N), jnp.float32),
        ],
        mesh=mesh,
        compiler_params=_SC_PARAMS,
        scratch_types=[
            pltpu.VMEM((N,), jnp.float32),
            pltpu.VMEM((N,), jnp.float32),
            pltpu.VMEM((EPT,), jnp.int32),
            pltpu.VMEM((EPT,), jnp.int32),
            pltpu.VMEM((EPT,), jnp.float32),
            pltpu.VMEM((N,), jnp.float32),
            pltpu.VMEM((128,), jnp.float32),
        ],
    )
    return f(sl, sr, eflat, zden)


def _scb_body(z_hbm, exin_hbm, ef_hbm, zrow_hbm,
              part_hbm,
              src_v, dst_v, ex_v, rows0_v, rows1_v, acc_sh,
              gsem0, gsem1, ssem):
    cid = lax.axis_index("c")
    sid = lax.axis_index("s")
    wid = sid * NC + (1 - cid)
    rows = (rows0_v, rows1_v)
    gsems = (gsem0, gsem1)

    pltpu.sync_copy(ef_hbm.at[0, wid], src_v)
    pltpu.sync_copy(ef_hbm.at[1, wid], dst_v)
    pltpu.sync_copy(exin_hbm.at[wid], ex_v)

    pltpu.sync_copy(zrow_hbm, rows0_v)
    base_row = sid * RPS
    for j in range(RPS // K):
        pltpu.sync_copy(rows0_v, acc_sh.at[pl.ds(base_row + j * K, K)])
    pltpu.sync_copy(rows0_v.at[pl.ds(0, RPS % K)],
                    acc_sh.at[pl.ds(base_row + (RPS // K) * K, RPS % K)])
    plsc.subcore_barrier()

    def _gather(c, b):
        pltpu.async_copy(z_hbm.at[src_v.at[pl.ds(c * K, K)]], rows[b],
                         gsems[b])

    _gather(0, 0)

    def pb(c2, carry):
        for b in range(2):
            c = 2 * c2 + b
            pltpu.make_async_copy(z_hbm.at[src_v.at[pl.ds(c * K, K)]],
                                  rows[b], gsems[b]).wait()

            @pl.when(c < CPT - 1)
            def _():
                _gather(c + 1, 1 - b)

            def row(jj, carry2):
                for u in range(4):
                    j = 4 * jj + u
                    b16 = plsc.load_gather(
                        ex_v, [jnp.full((16,), c * K + j, jnp.int32)])
                    for k in range(D // 16):
                        sl_ = pl.ds(k * 16, 16)
                        rows[b][j, sl_] = rows[b][j, sl_] * b16
                return carry2

            lax.fori_loop(0, K // 4, row, 0)
            for g in range(K // 16):
                d16 = dst_v[pl.ds(c * K + g * 16, 16)]
                pltpu.async_copy(rows[b].at[pl.ds(g * 16, 16)],
                                 acc_sh.at[d16], ssem, add=True)
            for g in range(K // 16):
                d16 = dst_v[pl.ds(c * K + g * 16, 16)]
                pltpu.make_async_copy(rows[b].at[pl.ds(g * 16, 16)],
                                      acc_sh.at[d16], ssem).wait()
        return carry

    lax.fori_loop(0, CPT // 2, pb, 0)

    plsc.subcore_barrier()
    pltpu.sync_copy(acc_sh.at[pl.ds(base_row, RPS)], part_hbm.at[cid, sid])


def _scb(z, ex, eflat, zrow):
    mesh = plsc.VectorSubcoreMesh(core_axis_name="c", subcore_axis_name="s")
    f = pl.kernel(
        _scb_body,
        out_type=jax.ShapeDtypeStruct((NC, NS, RPS, D), jnp.float32),
        mesh=mesh,
        compiler_params=_SC_PARAMS,
        scratch_types=[
            pltpu.VMEM((EPT,), jnp.int32),
            pltpu.VMEM((EPT,), jnp.int32),
            pltpu.VMEM((EPT,), jnp.float32),
            pltpu.VMEM((K, D), jnp.float32),
            pltpu.VMEM((K, D), jnp.float32),
            pltpu.VMEM_SHARED((N, D), jnp.float32),
            pltpu.SemaphoreType.DMA,
            pltpu.SemaphoreType.DMA,
            pltpu.SemaphoreType.DMA,
        ],
    )
    return f(z, ex, eflat, zrow)


def _tc2_body(p_ref, d_ref, out_ref):
    d = jnp.sum(d_ref[...], axis=0)
    s = p_ref[0] + p_ref[1]
    out_ref[...] = s * (1.0 / jnp.maximum(d, 1e-9))[:, None]


def _tc2(p, den):
    return pl.pallas_call(
        _tc2_body,
        out_shape=jax.ShapeDtypeStruct((N, D), jnp.float32),
    )(p, den)


@jax.jit
def kernel(h, edge_index, W_fc, a_w):
    z, sl, sr = _tc1(h, W_fc.T, a_w)
    ef = jnp.pad(edge_index, ((0, 0), (0, EPAD - E)))
    eflat = ef.reshape(2, NW, EPT)
    zden = jnp.zeros((N,), jnp.float32)
    zrow = jnp.zeros((K, D), jnp.float32)
    ex, den = _sca(sl, sr, eflat, zden)
    part = _scb(z, ex, eflat, zrow)
    return _tc2(part.reshape(NC, N, D), den)


# speedup vs baseline: 1.5854x; 1.5854x over previous
import jax
import jax.numpy as jnp
from jax import lax
from jax.experimental import pallas as pl
from jax.experimental.pallas import tpu as pltpu
from jax.experimental.pallas import tpu_sc as plsc

N = 10000
E = 320000
D = 128

NC = 2
NS = 16
NW = NC * NS
K = 64
CPT = 158
EPT = K * CPT
EPAD = EPT * NW
RPS = N // NS
NEG_SLOPE = 0.01

_SC_PARAMS = pltpu.CompilerParams(needs_layout_passes=False)


def _tc1_body(h_ref, wt_ref, aw_ref, z_ref, sl_ref, sr_ref):
    z = jnp.dot(h_ref[...], wt_ref[...], preferred_element_type=jnp.float32)
    z_ref[...] = z
    al = aw_ref[0, :D]
    ar = aw_ref[0, D:]
    sl_ref[...] = jnp.sum(z * al[None, :], axis=1)
    sr_ref[...] = jnp.sum(z * ar[None, :], axis=1)


def _tc1(h, wt, a_w):
    return pl.pallas_call(
        _tc1_body,
        out_shape=[
            jax.ShapeDtypeStruct((N, D), jnp.float32),
            jax.ShapeDtypeStruct((N,), jnp.float32),
            jax.ShapeDtypeStruct((N,), jnp.float32),
        ],
    )(h, wt, a_w)


def _sca_body(sl_hbm, sr_hbm, ef_hbm, zden_hbm,
              ex_hbm, den_hbm,
              sl_v, sr_v, src_v, dst_v, ex_v, den_v, mbuf_v):
    cid = lax.axis_index("c")
    sid = lax.axis_index("s")
    wid = sid * NC + cid

    pltpu.sync_copy(sl_hbm, sl_v)
    pltpu.sync_copy(sr_hbm, sr_v)
    pltpu.sync_copy(ef_hbm.at[0, wid], src_v)
    pltpu.sync_copy(ef_hbm.at[1, wid], dst_v)
    pltpu.sync_copy(zden_hbm, den_v)

    iota16 = lax.iota(jnp.int32, 16)

    def _mx(ref):
        def body(i, m):
            return jnp.maximum(m, ref[pl.ds(i * 16, 16)])
        m16 = lax.fori_loop(0, N // 16, body,
                            jnp.full((16,), -jnp.inf, jnp.float32))
        for sh in (8, 4, 2, 1):
            mbuf_v[pl.ds(0, 16)] = m16
            m16 = jnp.maximum(m16, plsc.load_gather(
                mbuf_v, [jnp.bitwise_xor(iota16, sh)]))
        return m16

    msum = _mx(sl_v) + _mx(sr_v)
    M = jnp.where(msum > 0, msum, NEG_SLOPE * msum)

    ebase = wid * EPT

    def pa(i, carry):
        s16 = src_v[pl.ds(i * 16, 16)]
        d16 = dst_v[pl.ds(i * 16, 16)]
        a = plsc.load_gather(sl_v, [s16])
        b = plsc.load_gather(sr_v, [d16])
        e = a + b
        e = jnp.where(e > 0, e, NEG_SLOPE * e)
        ex = jnp.exp(e - M)
        gid = ebase + i * 16 + iota16
        ex = jnp.where(gid < E, ex, 0.0)
        ex_v[pl.ds(i * 16, 16)] = ex
        plsc.addupdate_scatter(den_v, [d16], ex)
        return carry

    lax.fori_loop(0, EPT // 16, pa, 0)
    pltpu.sync_copy(ex_v, ex_hbm.at[wid])
    pltpu.sync_copy(den_v, den_hbm.at[wid])


def _sca(sl, sr, eflat, zden):
    mesh = plsc.VectorSubcoreMesh(core_axis_name="c", subcore_axis_name="s")
    f = pl.kernel(
        _sca_body,
        out_type=[
            jax.ShapeDtypeStruct((NW, EPT), jnp.float32),
            jax.ShapeDtypeStruct((NW, N), jnp.float32),
        ],
        mesh=mesh,
        compiler_params=_SC_PARAMS,
        scratch_types=[
            pltpu.VMEM((N,), jnp.float32),
            pltpu.VMEM((N,), jnp.float32),
            pltpu.VMEM((EPT,), jnp.int32),
            pltpu.VMEM((EPT,), jnp.int32),
            pltpu.VMEM((EPT,), jnp.float32),
            pltpu.VMEM((N,), jnp.float32),
            pltpu.VMEM((128,), jnp.float32),
        ],
    )
    return f(sl, sr, eflat, zden)


def _scb_body(z_hbm, exin_hbm, ef_hbm, zrow_hbm,
              part_hbm,
              src_v, dst_v, ex_v, rows0_v, rows1_v, acc_sh,
              gsem0, gsem1, ssem):
    cid = lax.axis_index("c")
    sid = lax.axis_index("s")
    wid = sid * NC + cid
    rows = (rows0_v, rows1_v)
    gsems = (gsem0, gsem1)

    pltpu.sync_copy(ef_hbm.at[0, wid], src_v)
    pltpu.sync_copy(ef_hbm.at[1, wid], dst_v)
    pltpu.sync_copy(exin_hbm.at[wid], ex_v)

    pltpu.sync_copy(zrow_hbm, rows0_v)
    base_row = sid * RPS
    for j in range(RPS // K):
        pltpu.sync_copy(rows0_v, acc_sh.at[pl.ds(base_row + j * K, K)])
    pltpu.sync_copy(rows0_v.at[pl.ds(0, RPS % K)],
                    acc_sh.at[pl.ds(base_row + (RPS // K) * K, RPS % K)])
    plsc.subcore_barrier()

    def _gather(c, b):
        pltpu.async_copy(z_hbm.at[src_v.at[pl.ds(c * K, K)]], rows[b],
                         gsems[b])

    _gather(0, 0)

    def pb(c2, carry):
        for b in range(2):
            c = 2 * c2 + b
            pltpu.make_async_copy(z_hbm.at[src_v.at[pl.ds(c * K, K)]],
                                  rows[b], gsems[b]).wait()

            @pl.when(c < CPT - 1)
            def _():
                _gather(c + 1, 1 - b)

            def row(jj, carry2):
                for u in range(4):
                    j = 4 * jj + u
                    b16 = plsc.load_gather(
                        ex_v, [jnp.full((16,), c * K + j, jnp.int32)])
                    for k in range(D // 16):
                        sl_ = pl.ds(k * 16, 16)
                        rows[b][j, sl_] = rows[b][j, sl_] * b16
                return carry2

            lax.fori_loop(0, K // 4, row, 0)
            for g in range(K // 16):
                d16 = dst_v[pl.ds(c * K + g * 16, 16)]
                pltpu.async_copy(rows[b].at[pl.ds(g * 16, 16)],
                                 acc_sh.at[d16], ssem, add=True)
            for g in range(K // 16):
                d16 = dst_v[pl.ds(c * K + g * 16, 16)]
                pltpu.make_async_copy(rows[b].at[pl.ds(g * 16, 16)],
                                      acc_sh.at[d16], ssem).wait()
        return carry

    lax.fori_loop(0, CPT // 2, pb, 0)

    plsc.subcore_barrier()
    pltpu.sync_copy(acc_sh.at[pl.ds(base_row, RPS)], part_hbm.at[cid, sid])


def _scb(z, ex, eflat, zrow):
    mesh = plsc.VectorSubcoreMesh(core_axis_name="c", subcore_axis_name="s")
    f = pl.kernel(
        _scb_body,
        out_type=jax.ShapeDtypeStruct((NC, NS, RPS, D), jnp.float32),
        mesh=mesh,
        compiler_params=_SC_PARAMS,
        scratch_types=[
            pltpu.VMEM((EPT,), jnp.int32),
            pltpu.VMEM((EPT,), jnp.int32),
            pltpu.VMEM((EPT,), jnp.float32),
            pltpu.VMEM((K, D), jnp.float32),
            pltpu.VMEM((K, D), jnp.float32),
            pltpu.VMEM_SHARED((N, D), jnp.float32),
            pltpu.SemaphoreType.DMA,
            pltpu.SemaphoreType.DMA,
            pltpu.SemaphoreType.DMA,
        ],
    )
    return f(z, ex, eflat, zrow)


def _tc2_body(p_ref, d_ref, out_ref):
    d = jnp.sum(d_ref[...], axis=0)
    s = p_ref[0] + p_ref[1]
    out_ref[...] = s * (1.0 / jnp.maximum(d, 1e-9))[:, None]


def _tc2(p, den):
    return pl.pallas_call(
        _tc2_body,
        out_shape=jax.ShapeDtypeStruct((N, D), jnp.float32),
    )(p, den)


@jax.jit
def kernel(h, edge_index, W_fc, a_w):
    z, sl, sr = _tc1(h, W_fc.T, a_w)
    padv = jnp.arange(EPAD - E, dtype=jnp.int32) % N
    ef = jnp.concatenate(
        [edge_index, jnp.stack([padv, padv])], axis=1)
    eflat = ef.reshape(2, NW, EPT)
    zden = jnp.zeros((N,), jnp.float32)
    zrow = jnp.zeros((K, D), jnp.float32)
    ex, den = _sca(sl, sr, eflat, zden)
    part = _scb(z, ex, eflat, zrow)
    return _tc2(part.reshape(NC, N, D), den)


# speedup vs baseline: 1.6428x; 1.0362x over previous
import jax
import jax.numpy as jnp
from jax import lax
from jax.experimental import pallas as pl
from jax.experimental.pallas import tpu as pltpu
from jax.experimental.pallas import tpu_sc as plsc

N = 10000
E = 320000
D = 128

NC = 2
NS = 16
NW = NC * NS
K = 128
CPT = 80
EPT = K * CPT
EPAD = EPT * NW
RPS = N // NS
NEG_SLOPE = 0.01

_SC_PARAMS = pltpu.CompilerParams(needs_layout_passes=False)


def _tc1_body(h_ref, wt_ref, aw_ref, z_ref, sl_ref, sr_ref):
    z = jnp.dot(h_ref[...], wt_ref[...], preferred_element_type=jnp.float32)
    z_ref[...] = z
    al = aw_ref[0, :D]
    ar = aw_ref[0, D:]
    sl_ref[...] = jnp.sum(z * al[None, :], axis=1)
    sr_ref[...] = jnp.sum(z * ar[None, :], axis=1)


def _tc1(h, wt, a_w):
    return pl.pallas_call(
        _tc1_body,
        out_shape=[
            jax.ShapeDtypeStruct((N, D), jnp.float32),
            jax.ShapeDtypeStruct((N,), jnp.float32),
            jax.ShapeDtypeStruct((N,), jnp.float32),
        ],
    )(h, wt, a_w)


def _sca_body(sl_hbm, sr_hbm, ef_hbm, zden_hbm,
              ex_hbm, den_hbm,
              sl_v, sr_v, src_v, dst_v, ex_v, den_v, mbuf_v):
    cid = lax.axis_index("c")
    sid = lax.axis_index("s")
    wid = sid * NC + cid

    pltpu.sync_copy(sl_hbm, sl_v)
    pltpu.sync_copy(sr_hbm, sr_v)
    pltpu.sync_copy(ef_hbm.at[0, wid], src_v)
    pltpu.sync_copy(ef_hbm.at[1, wid], dst_v)
    pltpu.sync_copy(zden_hbm, den_v)

    iota16 = lax.iota(jnp.int32, 16)

    def _mx(ref):
        def body(i, m):
            return jnp.maximum(m, ref[pl.ds(i * 16, 16)])
        m16 = lax.fori_loop(0, N // 16, body,
                            jnp.full((16,), -jnp.inf, jnp.float32))
        for sh in (8, 4, 2, 1):
            mbuf_v[pl.ds(0, 16)] = m16
            m16 = jnp.maximum(m16, plsc.load_gather(
                mbuf_v, [jnp.bitwise_xor(iota16, sh)]))
        return m16

    msum = _mx(sl_v) + _mx(sr_v)
    M = jnp.where(msum > 0, msum, NEG_SLOPE * msum)

    ebase = wid * EPT

    def pa(i, carry):
        s16 = src_v[pl.ds(i * 16, 16)]
        d16 = dst_v[pl.ds(i * 16, 16)]
        a = plsc.load_gather(sl_v, [s16])
        b = plsc.load_gather(sr_v, [d16])
        e = a + b
        e = jnp.where(e > 0, e, NEG_SLOPE * e)
        ex = jnp.exp(e - M)
        gid = ebase + i * 16 + iota16
        ex = jnp.where(gid < E, ex, 0.0)
        ex_v[pl.ds(i * 16, 16)] = ex
        plsc.addupdate_scatter(den_v, [d16], ex)
        return carry

    lax.fori_loop(0, EPT // 16, pa, 0)
    pltpu.sync_copy(ex_v, ex_hbm.at[wid])
    pltpu.sync_copy(den_v, den_hbm.at[wid])


def _sca(sl, sr, eflat, zden):
    mesh = plsc.VectorSubcoreMesh(core_axis_name="c", subcore_axis_name="s")
    f = pl.kernel(
        _sca_body,
        out_type=[
            jax.ShapeDtypeStruct((NW, EPT), jnp.float32),
            jax.ShapeDtypeStruct((NW, N), jnp.float32),
        ],
        mesh=mesh,
        compiler_params=_SC_PARAMS,
        scratch_types=[
            pltpu.VMEM((N,), jnp.float32),
            pltpu.VMEM((N,), jnp.float32),
            pltpu.VMEM((EPT,), jnp.int32),
            pltpu.VMEM((EPT,), jnp.int32),
            pltpu.VMEM((EPT,), jnp.float32),
            pltpu.VMEM((N,), jnp.float32),
            pltpu.VMEM((128,), jnp.float32),
        ],
    )
    return f(sl, sr, eflat, zden)


def _scb_body(z_hbm, exc_hbm, ef_hbm, dst3_hbm, zrow_hbm,
              part_hbm,
              src_v, rows0_v, rows1_v, dstb0_v, dstb1_v, exb0_v, exb1_v,
              acc_sh, gsem0, gsem1, isem0, isem1):
    cid = lax.axis_index("c")
    sid = lax.axis_index("s")
    wid = sid * NC + cid
    rows = (rows0_v, rows1_v)
    dstb = (dstb0_v, dstb1_v)
    exb = (exb0_v, exb1_v)
    gsems = (gsem0, gsem1)
    isems = (isem0, isem1)

    pltpu.sync_copy(ef_hbm.at[0, wid], src_v)

    pltpu.sync_copy(zrow_hbm, rows0_v)
    base_row = sid * RPS
    for j in range(RPS // K):
        pltpu.sync_copy(rows0_v, acc_sh.at[pl.ds(base_row + j * K, K)])
    pltpu.sync_copy(rows0_v.at[pl.ds(0, RPS % K)],
                    acc_sh.at[pl.ds(base_row + (RPS // K) * K, RPS % K)])
    plsc.subcore_barrier()

    def _fetch(c, b):
        pltpu.async_copy(z_hbm.at[src_v.at[pl.ds(c * K, K)]], rows[b],
                         gsems[b])
        pltpu.async_copy(dst3_hbm.at[wid, c], dstb[b], isems[b])
        pltpu.async_copy(exc_hbm.at[wid, c], exb[b], isems[b])

    def _drain(c, b):
        pltpu.make_async_copy(z_hbm.at[src_v.at[pl.ds(c * K, K)]],
                              rows[b], gsems[b]).wait()
        pltpu.make_async_copy(dst3_hbm.at[wid, c], dstb[b],
                              isems[b]).wait()
        pltpu.make_async_copy(exc_hbm.at[wid, c], exb[b], isems[b]).wait()

    _fetch(0, 0)

    def _process(c, b):
        _drain(c, b)

        @pl.when(c < CPT - 1)
        def _():
            _fetch(c + 1, 1 - b)

        def row(jj, carry2):
            for u in range(4):
                j = 4 * jj + u
                b16 = plsc.load_gather(
                    exb[b], [jnp.full((16,), j, jnp.int32)])
                for k in range(D // 16):
                    sl_ = pl.ds(k * 16, 16)
                    rows[b][j, sl_] = rows[b][j, sl_] * b16
            return carry2

        lax.fori_loop(0, K // 4, row, 0)
        pltpu.sync_copy(rows[b], acc_sh.at[dstb[b]], add=True)

    def pb(c2, carry):
        for b in range(2):
            _process(2 * c2 + b, b)
        return carry

    lax.fori_loop(0, CPT // 2, pb, 0)

    plsc.subcore_barrier()
    pltpu.sync_copy(acc_sh.at[pl.ds(base_row, RPS)], part_hbm.at[cid, sid])


def _scb(z, exc, eflat, dst3, zrow):
    mesh = plsc.VectorSubcoreMesh(core_axis_name="c", subcore_axis_name="s")
    f = pl.kernel(
        _scb_body,
        out_type=jax.ShapeDtypeStruct((NC, NS, RPS, D), jnp.float32),
        mesh=mesh,
        compiler_params=_SC_PARAMS,
        scratch_types=[
            pltpu.VMEM((EPT,), jnp.int32),
            pltpu.VMEM((K, D), jnp.float32),
            pltpu.VMEM((K, D), jnp.float32),
            pltpu.VMEM((K,), jnp.int32),
            pltpu.VMEM((K,), jnp.int32),
            pltpu.VMEM((K,), jnp.float32),
            pltpu.VMEM((K,), jnp.float32),
            pltpu.VMEM_SHARED((N, D), jnp.float32),
            pltpu.SemaphoreType.DMA,
            pltpu.SemaphoreType.DMA,
            pltpu.SemaphoreType.DMA,
            pltpu.SemaphoreType.DMA,
        ],
    )
    return f(z, exc, eflat, dst3, zrow)


def _tc2_body(p_ref, d_ref, out_ref):
    d = jnp.sum(d_ref[...], axis=0)
    s = p_ref[0] + p_ref[1]
    out_ref[...] = s * (1.0 / jnp.maximum(d, 1e-9))[:, None]


def _tc2(p, den):
    return pl.pallas_call(
        _tc2_body,
        out_shape=jax.ShapeDtypeStruct((N, D), jnp.float32),
    )(p, den)


@jax.jit
def kernel(h, edge_index, W_fc, a_w):
    z, sl, sr = _tc1(h, W_fc.T, a_w)
    padv = jnp.arange(EPAD - E, dtype=jnp.int32) % N
    ef = jnp.concatenate(
        [edge_index, jnp.stack([padv, padv])], axis=1)
    eflat = ef.reshape(2, NW, EPT)
    zden = jnp.zeros((N,), jnp.float32)
    zrow = jnp.zeros((K, D), jnp.float32)
    dst3 = ef[1].reshape(NW, CPT, K)
    ex, den = _sca(sl, sr, eflat, zden)
    part = _scb(z, ex.reshape(NW, CPT, K), eflat, dst3, zrow)
    return _tc2(part.reshape(NC, N, D), den)


# speedup vs baseline: 1.7323x; 1.0545x over previous
import jax
import jax.numpy as jnp
from jax import lax
from jax.experimental import pallas as pl
from jax.experimental.pallas import tpu as pltpu
from jax.experimental.pallas import tpu_sc as plsc

N = 10000
E = 320000
D = 128

NC = 2
NS = 16
NW = NC * NS
K = 96
CPT = 105
EPT = K * CPT
EPAD = EPT * NW
RPS = N // NS
NEG_SLOPE = 0.01

_SC_PARAMS = pltpu.CompilerParams(needs_layout_passes=False)


def _tc1_body(h_ref, wt_ref, aw_ref, z_ref, sl_ref, sr_ref):
    z = jnp.dot(h_ref[...], wt_ref[...], preferred_element_type=jnp.float32)
    z_ref[...] = z
    al = aw_ref[0, :D]
    ar = aw_ref[0, D:]
    sl_ref[...] = jnp.sum(z * al[None, :], axis=1)
    sr_ref[...] = jnp.sum(z * ar[None, :], axis=1)


def _tc1(h, wt, a_w):
    return pl.pallas_call(
        _tc1_body,
        out_shape=[
            jax.ShapeDtypeStruct((N, D), jnp.float32),
            jax.ShapeDtypeStruct((N,), jnp.float32),
            jax.ShapeDtypeStruct((N,), jnp.float32),
        ],
    )(h, wt, a_w)


def _sca_body(sl_hbm, sr_hbm, ef_hbm, zden_hbm,
              ex_hbm, den_hbm,
              sl_v, sr_v, src_v, dst_v, ex_v, den_v, mbuf_v):
    cid = lax.axis_index("c")
    sid = lax.axis_index("s")
    wid = sid * NC + cid

    pltpu.sync_copy(sl_hbm, sl_v)
    pltpu.sync_copy(sr_hbm, sr_v)
    pltpu.sync_copy(ef_hbm.at[0, wid], src_v)
    pltpu.sync_copy(ef_hbm.at[1, wid], dst_v)
    pltpu.sync_copy(zden_hbm, den_v)

    iota16 = lax.iota(jnp.int32, 16)

    def _mx(ref):
        def body(i, m):
            return jnp.maximum(m, ref[pl.ds(i * 16, 16)])
        m16 = lax.fori_loop(0, N // 16, body,
                            jnp.full((16,), -jnp.inf, jnp.float32))
        for sh in (8, 4, 2, 1):
            mbuf_v[pl.ds(0, 16)] = m16
            m16 = jnp.maximum(m16, plsc.load_gather(
                mbuf_v, [jnp.bitwise_xor(iota16, sh)]))
        return m16

    msum = _mx(sl_v) + _mx(sr_v)
    M = jnp.where(msum > 0, msum, NEG_SLOPE * msum)

    ebase = wid * EPT

    def pa(i, carry):
        s16 = src_v[pl.ds(i * 16, 16)]
        d16 = dst_v[pl.ds(i * 16, 16)]
        a = plsc.load_gather(sl_v, [s16])
        b = plsc.load_gather(sr_v, [d16])
        e = a + b
        e = jnp.where(e > 0, e, NEG_SLOPE * e)
        ex = jnp.exp(e - M)
        gid = ebase + i * 16 + iota16
        ex = jnp.where(gid < E, ex, 0.0)
        ex_v[pl.ds(i * 16, 16)] = ex
        plsc.addupdate_scatter(den_v, [d16], ex)
        return carry

    lax.fori_loop(0, EPT // 16, pa, 0)
    pltpu.sync_copy(ex_v, ex_hbm.at[wid])
    pltpu.sync_copy(den_v, den_hbm.at[wid])


def _sca(sl, sr, eflat, zden):
    mesh = plsc.VectorSubcoreMesh(core_axis_name="c", subcore_axis_name="s")
    f = pl.kernel(
        _sca_body,
        out_type=[
            jax.ShapeDtypeStruct((NW, EPT), jnp.float32),
            jax.ShapeDtypeStruct((NW, N), jnp.float32),
        ],
        mesh=mesh,
        compiler_params=_SC_PARAMS,
        scratch_types=[
            pltpu.VMEM((N,), jnp.float32),
            pltpu.VMEM((N,), jnp.float32),
            pltpu.VMEM((EPT,), jnp.int32),
            pltpu.VMEM((EPT,), jnp.int32),
            pltpu.VMEM((EPT,), jnp.float32),
            pltpu.VMEM((N,), jnp.float32),
            pltpu.VMEM((128,), jnp.float32),
        ],
    )
    return f(sl, sr, eflat, zden)


def _scb_body(z_hbm, exc_hbm, ef_hbm, dst3_hbm, zrow_hbm,
              part_hbm,
              src_v, rows0_v, rows1_v, rows2_v,
              dstb0_v, dstb1_v, dstb2_v, exb0_v, exb1_v, exb2_v,
              acc_sh, gsem0, gsem1, gsem2, isem0, isem1, isem2,
              ssem0, ssem1, ssem2):
    cid = lax.axis_index("c")
    sid = lax.axis_index("s")
    wid = sid * NC + cid
    rows = (rows0_v, rows1_v, rows2_v)
    dstb = (dstb0_v, dstb1_v, dstb2_v)
    exb = (exb0_v, exb1_v, exb2_v)
    gsems = (gsem0, gsem1, gsem2)
    isems = (isem0, isem1, isem2)
    ssems = (ssem0, ssem1, ssem2)

    pltpu.sync_copy(ef_hbm.at[0, wid], src_v)

    pltpu.sync_copy(zrow_hbm, rows0_v)
    base_row = sid * RPS
    for j in range(RPS // K):
        pltpu.sync_copy(rows0_v, acc_sh.at[pl.ds(base_row + j * K, K)])
    pltpu.sync_copy(rows0_v.at[pl.ds(0, RPS % K)],
                    acc_sh.at[pl.ds(base_row + (RPS // K) * K, RPS % K)])
    plsc.subcore_barrier()

    def _fetch(c, b):
        pltpu.async_copy(z_hbm.at[src_v.at[pl.ds(c * K, K)]], rows[b],
                         gsems[b])
        pltpu.async_copy(dst3_hbm.at[wid, c], dstb[b], isems[b])
        pltpu.async_copy(exc_hbm.at[wid, c], exb[b], isems[b])

    def _drain(c, b):
        pltpu.make_async_copy(z_hbm.at[src_v.at[pl.ds(c * K, K)]],
                              rows[b], gsems[b]).wait()
        pltpu.make_async_copy(dst3_hbm.at[wid, c], dstb[b],
                              isems[b]).wait()
        pltpu.make_async_copy(exc_hbm.at[wid, c], exb[b], isems[b]).wait()

    def _drain_scatter(b):
        pltpu.make_async_copy(rows[b], acc_sh.at[dstb[b]], ssems[b]).wait()

    _fetch(0, 0)

    def _process(c, b):
        _drain(c, b)

        @pl.when(c >= 2)
        def _():
            _drain_scatter((b + 1) % 3)

        @pl.when(c < CPT - 1)
        def _():
            _fetch(c + 1, (b + 1) % 3)

        def row(jj, carry2):
            for u in range(4):
                j = 4 * jj + u
                b16 = plsc.load_gather(
                    exb[b], [jnp.full((16,), j, jnp.int32)])
                for k in range(D // 16):
                    sl_ = pl.ds(k * 16, 16)
                    rows[b][j, sl_] = rows[b][j, sl_] * b16
            return carry2

        lax.fori_loop(0, K // 4, row, 0)
        pltpu.async_copy(rows[b], acc_sh.at[dstb[b]], ssems[b], add=True)

    def pb(c3, carry):
        for u in range(3):
            _process(3 * c3 + u, u)
        return carry

    lax.fori_loop(0, CPT // 3, pb, 0)
    _drain_scatter((CPT - 2) % 3)
    _drain_scatter((CPT - 1) % 3)

    plsc.subcore_barrier()
    pltpu.sync_copy(acc_sh.at[pl.ds(base_row, RPS)], part_hbm.at[cid, sid])


def _scb(z, exc, eflat, dst3, zrow):
    mesh = plsc.VectorSubcoreMesh(core_axis_name="c", subcore_axis_name="s")
    f = pl.kernel(
        _scb_body,
        out_type=jax.ShapeDtypeStruct((NC, NS, RPS, D), jnp.float32),
        mesh=mesh,
        compiler_params=_SC_PARAMS,
        scratch_types=[
            pltpu.VMEM((EPT,), jnp.int32),
            pltpu.VMEM((K, D), jnp.float32),
            pltpu.VMEM((K, D), jnp.float32),
            pltpu.VMEM((K, D), jnp.float32),
            pltpu.VMEM((K,), jnp.int32),
            pltpu.VMEM((K,), jnp.int32),
            pltpu.VMEM((K,), jnp.int32),
            pltpu.VMEM((K,), jnp.float32),
            pltpu.VMEM((K,), jnp.float32),
            pltpu.VMEM((K,), jnp.float32),
            pltpu.VMEM_SHARED((N, D), jnp.float32),
            pltpu.SemaphoreType.DMA,
            pltpu.SemaphoreType.DMA,
            pltpu.SemaphoreType.DMA,
            pltpu.SemaphoreType.DMA,
            pltpu.SemaphoreType.DMA,
            pltpu.SemaphoreType.DMA,
            pltpu.SemaphoreType.DMA,
            pltpu.SemaphoreType.DMA,
            pltpu.SemaphoreType.DMA,
        ],
    )
    return f(z, exc, eflat, dst3, zrow)


def _tc2_body(p_ref, d_ref, out_ref):
    d = jnp.sum(d_ref[...], axis=0)
    s = p_ref[0] + p_ref[1]
    out_ref[...] = s * (1.0 / jnp.maximum(d, 1e-9))[:, None]


def _tc2(p, den):
    return pl.pallas_call(
        _tc2_body,
        out_shape=jax.ShapeDtypeStruct((N, D), jnp.float32),
    )(p, den)


@jax.jit
def kernel(h, edge_index, W_fc, a_w):
    z, sl, sr = _tc1(h, W_fc.T, a_w)
    padv = jnp.arange(EPAD - E, dtype=jnp.int32) % N
    ef = jnp.concatenate(
        [edge_index, jnp.stack([padv, padv])], axis=1)
    eflat = ef.reshape(2, NW, EPT)
    zden = jnp.zeros((N,), jnp.float32)
    zrow = jnp.zeros((K, D), jnp.float32)
    dst3 = ef[1].reshape(NW, CPT, K)
    ex, den = _sca(sl, sr, eflat, zden)
    part = _scb(z, ex.reshape(NW, CPT, K), eflat, dst3, zrow)
    return _tc2(part.reshape(NC, N, D), den)


# speedup vs baseline: 1.7332x; 1.0005x over previous
import jax
import jax.numpy as jnp
from jax import lax
from jax.experimental import pallas as pl
from jax.experimental.pallas import tpu as pltpu
from jax.experimental.pallas import tpu_sc as plsc

N = 10000
E = 320000
D = 128

NC = 2
NS = 16
NW = NC * NS
K = 96
CPT = 105
EPT = K * CPT
EPAD = EPT * NW
RPS = N // NS
NEG_SLOPE = 0.01

_SC_PARAMS = pltpu.CompilerParams(needs_layout_passes=False)


def _tc1_body(h_ref, wt_ref, aw_ref, z_ref, sl_ref, sr_ref):
    z = jnp.dot(h_ref[...], wt_ref[...], preferred_element_type=jnp.float32)
    z_ref[...] = z
    al = aw_ref[0, :D]
    ar = aw_ref[0, D:]
    sl_ref[...] = jnp.sum(z * al[None, :], axis=1)
    sr_ref[...] = jnp.sum(z * ar[None, :], axis=1)


def _tc1(h, wt, a_w):
    return pl.pallas_call(
        _tc1_body,
        out_shape=[
            jax.ShapeDtypeStruct((N, D), jnp.float32),
            jax.ShapeDtypeStruct((N,), jnp.float32),
            jax.ShapeDtypeStruct((N,), jnp.float32),
        ],
    )(h, wt, a_w)


def _sca_body(sl_hbm, sr_hbm, ef_hbm, zden_hbm,
              ex_hbm, den_hbm,
              sl_v, sr_v, src_v, dst_v, ex_v, den_v, mbuf_v):
    cid = lax.axis_index("c")
    sid = lax.axis_index("s")
    wid = sid * NC + cid

    pltpu.sync_copy(sl_hbm, sl_v)
    pltpu.sync_copy(sr_hbm, sr_v)
    pltpu.sync_copy(ef_hbm.at[0, wid], src_v)
    pltpu.sync_copy(ef_hbm.at[1, wid], dst_v)
    pltpu.sync_copy(zden_hbm, den_v)

    iota16 = lax.iota(jnp.int32, 16)

    def _mx(ref):
        def body(i, m):
            return jnp.maximum(m, ref[pl.ds(i * 16, 16)])
        m16 = lax.fori_loop(0, N // 16, body,
                            jnp.full((16,), -jnp.inf, jnp.float32))
        for sh in (8, 4, 2, 1):
            mbuf_v[pl.ds(0, 16)] = m16
            m16 = jnp.maximum(m16, plsc.load_gather(
                mbuf_v, [jnp.bitwise_xor(iota16, sh)]))
        return m16

    msum = _mx(sl_v) + _mx(sr_v)
    M = jnp.where(msum > 0, msum, NEG_SLOPE * msum)

    ebase = wid * EPT

    def pa(i, carry):
        s16 = src_v[pl.ds(i * 16, 16)]
        d16 = dst_v[pl.ds(i * 16, 16)]
        a = plsc.load_gather(sl_v, [s16])
        b = plsc.load_gather(sr_v, [d16])
        e = a + b
        e = jnp.where(e > 0, e, NEG_SLOPE * e)
        ex = jnp.exp(e - M)
        gid = ebase + i * 16 + iota16
        ex = jnp.where(gid < E, ex, 0.0)
        ex_v[pl.ds(i * 16, 16)] = ex
        plsc.addupdate_scatter(den_v, [d16], ex)
        return carry

    lax.fori_loop(0, EPT // 16, pa, 0)
    pltpu.sync_copy(ex_v, ex_hbm.at[wid])
    pltpu.sync_copy(den_v, den_hbm.at[wid])


def _sca(sl, sr, eflat, zden):
    mesh = plsc.VectorSubcoreMesh(core_axis_name="c", subcore_axis_name="s")
    f = pl.kernel(
        _sca_body,
        out_type=[
            jax.ShapeDtypeStruct((NW, EPT), jnp.float32),
            jax.ShapeDtypeStruct((NW, N), jnp.float32),
        ],
        mesh=mesh,
        compiler_params=_SC_PARAMS,
        scratch_types=[
            pltpu.VMEM((N,), jnp.float32),
            pltpu.VMEM((N,), jnp.float32),
            pltpu.VMEM((EPT,), jnp.int32),
            pltpu.VMEM((EPT,), jnp.int32),
            pltpu.VMEM((EPT,), jnp.float32),
            pltpu.VMEM((N,), jnp.float32),
            pltpu.VMEM((128,), jnp.float32),
        ],
    )
    return f(sl, sr, eflat, zden)


def _scb_body(z_hbm, exc_hbm, ef_hbm, dst3_hbm, zrow_hbm,
              part_hbm,
              src_v, rows0_v, rows1_v, rows2_v,
              dstb0_v, dstb1_v, dstb2_v, exb0_v, exb1_v, exb2_v,
              acc_sh, gsem0, gsem1, gsem2, isem0, isem1, isem2,
              ssem0, ssem1, ssem2):
    cid = lax.axis_index("c")
    sid = lax.axis_index("s")
    wid = sid * NC + cid
    rows = (rows0_v, rows1_v, rows2_v)
    dstb = (dstb0_v, dstb1_v, dstb2_v)
    exb = (exb0_v, exb1_v, exb2_v)
    gsems = (gsem0, gsem1, gsem2)
    isems = (isem0, isem1, isem2)
    ssems = (ssem0, ssem1, ssem2)

    pltpu.sync_copy(ef_hbm.at[0, wid], src_v)

    pltpu.sync_copy(zrow_hbm, rows0_v)
    base_row = sid * RPS
    for j in range(RPS // K):
        pltpu.sync_copy(rows0_v, acc_sh.at[pl.ds(base_row + j * K, K)])
    pltpu.sync_copy(rows0_v.at[pl.ds(0, RPS % K)],
                    acc_sh.at[pl.ds(base_row + (RPS // K) * K, RPS % K)])
    plsc.subcore_barrier()

    def _fetch(c, b):
        pltpu.async_copy(z_hbm.at[src_v.at[pl.ds(c * K, K)]], rows[b],
                         gsems[b])
        pltpu.async_copy(dst3_hbm.at[wid, c], dstb[b], isems[b])
        pltpu.async_copy(exc_hbm.at[wid, c], exb[b], isems[b])

    def _drain(c, b):
        pltpu.make_async_copy(z_hbm.at[src_v.at[pl.ds(c * K, K)]],
                              rows[b], gsems[b]).wait()
        pltpu.make_async_copy(dst3_hbm.at[wid, c], dstb[b],
                              isems[b]).wait()
        pltpu.make_async_copy(exc_hbm.at[wid, c], exb[b], isems[b]).wait()

    def _drain_scatter(b):
        pltpu.make_async_copy(rows[b], acc_sh.at[dstb[b]], ssems[b]).wait()

    _fetch(0, 0)

    def _process(c, b):
        _drain(c, b)

        @pl.when(c >= 2)
        def _():
            _drain_scatter((b + 1) % 3)

        @pl.when(c < CPT - 1)
        def _():
            _fetch(c + 1, (b + 1) % 3)

        def row(jj, carry2):
            for u in range(8):
                j = 8 * jj + u
                b16 = plsc.load_gather(
                    exb[b], [jnp.full((16,), j, jnp.int32)])
                for k in range(D // 16):
                    sl_ = pl.ds(k * 16, 16)
                    rows[b][j, sl_] = rows[b][j, sl_] * b16
            return carry2

        lax.fori_loop(0, K // 8, row, 0)
        pltpu.async_copy(rows[b], acc_sh.at[dstb[b]], ssems[b], add=True)

    def pb(c3, carry):
        for u in range(3):
            _process(3 * c3 + u, u)
        return carry

    lax.fori_loop(0, CPT // 3, pb, 0)
    _drain_scatter((CPT - 2) % 3)
    _drain_scatter((CPT - 1) % 3)

    plsc.subcore_barrier()
    pltpu.sync_copy(acc_sh.at[pl.ds(base_row, RPS)], part_hbm.at[cid, sid])


def _scb(z, exc, eflat, dst3, zrow):
    mesh = plsc.VectorSubcoreMesh(core_axis_name="c", subcore_axis_name="s")
    f = pl.kernel(
        _scb_body,
        out_type=jax.ShapeDtypeStruct((NC, NS, RPS, D), jnp.float32),
        mesh=mesh,
        compiler_params=_SC_PARAMS,
        scratch_types=[
            pltpu.VMEM((EPT,), jnp.int32),
            pltpu.VMEM((K, D), jnp.float32),
            pltpu.VMEM((K, D), jnp.float32),
            pltpu.VMEM((K, D), jnp.float32),
            pltpu.VMEM((K,), jnp.int32),
            pltpu.VMEM((K,), jnp.int32),
            pltpu.VMEM((K,), jnp.int32),
            pltpu.VMEM((K,), jnp.float32),
            pltpu.VMEM((K,), jnp.float32),
            pltpu.VMEM((K,), jnp.float32),
            pltpu.VMEM_SHARED((N, D), jnp.float32),
            pltpu.SemaphoreType.DMA,
            pltpu.SemaphoreType.DMA,
            pltpu.SemaphoreType.DMA,
            pltpu.SemaphoreType.DMA,
            pltpu.SemaphoreType.DMA,
            pltpu.SemaphoreType.DMA,
            pltpu.SemaphoreType.DMA,
            pltpu.SemaphoreType.DMA,
            pltpu.SemaphoreType.DMA,
        ],
    )
    return f(z, exc, eflat, dst3, zrow)


def _tc2_body(p_ref, d_ref, out_ref):
    d = jnp.sum(d_ref[...], axis=0)
    s = p_ref[0] + p_ref[1]
    out_ref[...] = s * (1.0 / jnp.maximum(d, 1e-9))[:, None]


def _tc2(p, den):
    return pl.pallas_call(
        _tc2_body,
        out_shape=jax.ShapeDtypeStruct((N, D), jnp.float32),
    )(p, den)


@jax.jit
def kernel(h, edge_index, W_fc, a_w):
    z, sl, sr = _tc1(h, W_fc.T, a_w)
    padv = jnp.arange(EPAD - E, dtype=jnp.int32) % N
    ef = jnp.concatenate(
        [edge_index, jnp.stack([padv, padv])], axis=1)
    eflat = ef.reshape(2, NW, EPT)
    zden = jnp.zeros((N,), jnp.float32)
    zrow = jnp.zeros((K, D), jnp.float32)
    dst3 = ef[1].reshape(NW, CPT, K)
    ex, den = _sca(sl, sr, eflat, zden)
    part = _scb(z, ex.reshape(NW, CPT, K), eflat, dst3, zrow)
    return _tc2(part.reshape(NC, N, D), den)
